# Initial kernel scaffold; baseline (speedup 1.0000x reference)
import jax
import jax.numpy as jnp
from jax.experimental import pallas as pl


def kernel(x, table_1, table_2, table_3, table_4, table_5, table_6, table_7):
    raise NotImplementedError("write your pallas kernel here")


# speedup vs baseline: 1.4083x; 1.4083x over previous
import functools

import jax
import jax.numpy as jnp
from jax import lax
from jax.experimental import pallas as pl
from jax.experimental.pallas import tpu as pltpu
from jax.experimental.pallas import tpu_sc as plsc

EMBED = 16
NDEPTH = 7
CHUNK = 128
NWORKERS = 32
LANES = 16
CLIP_MAX = 1.0 - 1e-6


@functools.lru_cache(maxsize=None)
def _build(n_points):
    nchunks = -(-n_points // CHUNK)
    tail = n_points - (nchunks - 1) * CHUNK
    kpw = -(-nchunks // NWORKERS)

    mesh = plsc.VectorSubcoreMesh(
        core_axis_name="c", subcore_axis_name="s", num_cores=2, num_subcores=16
    )
    scratch = (
        [pltpu.VMEM((CHUNK,), jnp.float32) for _ in range(3)]
        + [pltpu.VMEM((CHUNK,), jnp.int32) for _ in range(NDEPTH)]
        + [pltpu.VMEM((CHUNK, EMBED), jnp.float32) for _ in range(NDEPTH)]
        + [pltpu.SemaphoreType.DMA]
    )

    @functools.partial(
        pl.kernel,
        out_type=jax.ShapeDtypeStruct((n_points, NDEPTH * EMBED), jnp.float32),
        mesh=mesh,
        scratch_types=scratch,
        compiler_params=pltpu.CompilerParams(use_tc_tiling_on_sc=False),
    )
    def grid_kernel(x0, x1, x2, t1, t2, t3, t4, t5, t6, t7, out, *sc):
        xs = (x0, x1, x2)
        xv = sc[0:3]
        iv = sc[3:3 + NDEPTH]
        rv = sc[3 + NDEPTH:3 + 2 * NDEPTH]
        sem = sc[-1]
        tabs = (t1, t2, t3, t4, t5, t6, t7)
        wid = lax.axis_index("s") * 2 + lax.axis_index("c")

        @pl.loop(0, kpw)
        def _chunk_loop(kk):
            ci = wid + kk * NWORKERS

            @pl.when(ci < nchunks)
            def _():
                base = ci * CHUNK
                for c in range(3):
                    pltpu.sync_copy(xs[c].at[pl.ds(base, CHUNK)], xv[c])

                @pl.loop(0, CHUNK // LANES)
                def _idx_loop(j):
                    sl = pl.ds(j * LANES, LANES)
                    sx = [
                        jnp.minimum(jnp.maximum(xv[c][sl] * 0.5 + 0.5, 0.0), CLIP_MAX)
                        for c in range(3)
                    ]
                    px, py, pz = sx
                    for d in range(NDEPTH):
                        px = px * 2.0
                        py = py * 2.0
                        pz = pz * 2.0
                        dd = 1 << (d + 1)
                        cx = px.astype(jnp.int32)
                        cy = py.astype(jnp.int32)
                        cz = pz.astype(jnp.int32)
                        iv[d][sl] = cx + cy * dd + cz * (dd * dd)

                cps = [
                    pltpu.async_copy(tabs[d].at[iv[d]], rv[d], sem)
                    for d in range(NDEPTH)
                ]
                for cp in cps:
                    cp.wait()

                if tail == CHUNK:
                    for d in range(NDEPTH):
                        pltpu.sync_copy(
                            rv[d],
                            out.at[pl.ds(base, CHUNK), pl.ds(d * EMBED, EMBED)],
                        )
                else:
                    @pl.when(ci < nchunks - 1)
                    def _():
                        for d in range(NDEPTH):
                            pltpu.sync_copy(
                                rv[d],
                                out.at[pl.ds(base, CHUNK), pl.ds(d * EMBED, EMBED)],
                            )

                    @pl.when(ci == nchunks - 1)
                    def _():
                        for d in range(NDEPTH):
                            pltpu.sync_copy(
                                rv[d].at[pl.ds(0, tail), :],
                                out.at[pl.ds(base, tail), pl.ds(d * EMBED, EMBED)],
                            )

    return grid_kernel


def kernel(x, table_1, table_2, table_3, table_4, table_5, table_6, table_7):
    n = x.shape[0]
    nchunks = -(-n // CHUNK)
    npad = nchunks * CHUNK
    xp = jnp.pad(x, ((0, npad - n), (0, 0)))
    x0 = xp[:, 0]
    x1 = xp[:, 1]
    x2 = xp[:, 2]
    fn = _build(n)
    return fn(x0, x1, x2, table_1, table_2, table_3, table_4,
              table_5, table_6, table_7)
